# baseline (speedup 1.0000x reference)
import jax
import jax.numpy as jnp
from jax import lax
from jax.experimental import pallas as pl
from jax.experimental.pallas import tpu as pltpu
from jax.experimental.pallas import tpu_sc as plsc

N_NODES_C = 10000
N_EDGES_C = 320000
D_FEAT_C = 128
N_HOPS_C = 3

HALF = D_FEAT_C // 2
N_SUBCORES = 16
CHUNK = 128
N_CHUNKS = 160
EDGES_PER_TILE = N_CHUNKS * CHUNK
NBUF = 5
PAD_COL = N_NODES_C
N_ACC = N_NODES_C + 16
W_SMALL = 624
W_LAST = N_NODES_C - 15 * W_SMALL


def _sc_body(embed, packed_hbm, trend_hbm, zeros_hbm,
             out2d, cur,
             acc, gbufs, pbufs, tbufs,
             sem_idx, sem_g, sem_sc):
    c = lax.axis_index("c")
    s = lax.axis_index("s")
    row_off = c * N_NODES_C
    fcol = c * HALF

    nr0 = s * W_SMALL
    nrows_small = W_SMALL

    def idx_start(q, b):
        chunk_id = s * N_CHUNKS + q
        pltpu.make_async_copy(packed_hbm.at[chunk_id],
                              pbufs.at[b], sem_idx.at[b]).start()
        base = pl.multiple_of(s * EDGES_PER_TILE + q * CHUNK, 8)
        pltpu.make_async_copy(trend_hbm.at[pl.ds(base, CHUNK)],
                              tbufs.at[b], sem_idx.at[b]).start()

    def idx_wait(b):
        pltpu.make_async_copy(packed_hbm.at[0],
                              pbufs.at[b], sem_idx.at[b]).wait()
        pltpu.make_async_copy(trend_hbm.at[pl.ds(0, CHUNK)],
                              tbufs.at[b], sem_idx.at[b]).wait()

    def rowfix(b):
        for v in range(CHUNK // 16):
            pbufs[b, pl.ds(v * 16, 16)] = (
                pbufs[b, pl.ds(v * 16, 16)] + row_off)

    def gather_start(b):
        pltpu.make_async_copy(cur.at[pbufs.at[b, pl.ds(0, CHUNK)]],
                              gbufs.at[b], sem_g.at[b]).start()

    def gather_wait(b):
        pltpu.make_async_copy(cur.at[pbufs.at[b, pl.ds(0, CHUNK)]],
                              gbufs.at[b], sem_g.at[b]).wait()

    def scat_start(b):
        pltpu.make_async_copy(gbufs.at[b],
                              acc.at[pbufs.at[b, pl.ds(CHUNK, CHUNK)]],
                              sem_sc.at[b]).start(add=True)

    def scat_wait(b):
        pltpu.make_async_copy(gbufs.at[b],
                              acc.at[pbufs.at[b, pl.ds(CHUNK, CHUNK)]],
                              sem_sc.at[b]).wait()

    def make_scale(b):
        def scale_group(g, _):
            tv16 = tbufs[b, pl.ds(g * 16, 16)]
            for sub in range(2):
                e0 = g * 16 + sub * 8
                tvs = [jnp.full((16,), tv16[sub * 8 + l], jnp.float32)
                       for l in range(8)]
                vs = [[gbufs[b, e0 + l, pl.ds(jj * 16, 16)]
                       for jj in range(HALF // 16)] for l in range(8)]
                for l in range(8):
                    for jj in range(HALF // 16):
                        gbufs[b, e0 + l, pl.ds(jj * 16, 16)] = (
                            vs[l][jj] * tvs[l])
            return 0
        return scale_group

    scales = [make_scale(b) for b in range(NBUF)]

    @pl.when(s < 15)
    def _():
        pltpu.sync_copy(embed.at[pl.ds(nr0, W_SMALL), pl.ds(fcol, HALF)],
                        cur.at[pl.ds(row_off + nr0, W_SMALL)])
        pltpu.sync_copy(embed.at[pl.ds(nr0, W_SMALL), pl.ds(fcol, HALF)],
                        out2d.at[pl.ds(nr0, W_SMALL), pl.ds(fcol, HALF)])

    @pl.when(s == 15)
    def _():
        pltpu.sync_copy(
            embed.at[pl.ds(15 * W_SMALL, W_LAST), pl.ds(fcol, HALF)],
            cur.at[pl.ds(row_off + 15 * W_SMALL, W_LAST)])
        pltpu.sync_copy(
            embed.at[pl.ds(15 * W_SMALL, W_LAST), pl.ds(fcol, HALF)],
            out2d.at[pl.ds(15 * W_SMALL, W_LAST), pl.ds(fcol, HALF)])

    plsc.subcore_barrier()

    def hop_body(h, _):
        idx_start(0, 0)
        idx_start(1, 1)

        @pl.when(s < 15)
        def _():
            pltpu.sync_copy(zeros_hbm.at[pl.ds(0, W_SMALL)],
                            acc.at[pl.ds(s * W_SMALL, W_SMALL)])

        @pl.when(s == 15)
        def _():
            pltpu.sync_copy(zeros_hbm, acc.at[pl.ds(15 * W_SMALL, W_LAST)])

        idx_wait(0)
        rowfix(0)
        gather_start(0)
        plsc.subcore_barrier()


        def outer(jo, _):
            for b in range(NBUF):
                q = jo * NBUF + b
                bp = (b + 2) % NBUF
                bn = (b + 1) % NBUF

                @pl.when(q + 2 < N_CHUNKS)
                def _():
                    @pl.when(q + 2 >= NBUF)
                    def _():
                        scat_wait(bp)
                    idx_start(q + 2, bp)

                @pl.when(q + 1 < N_CHUNKS)
                def _():
                    idx_wait(bn)
                    rowfix(bn)
                    gather_start(bn)

                gather_wait(b)
                lax.fori_loop(0, CHUNK // 16, scales[b], 0)
                scat_start(b)
            return 0

        lax.fori_loop(0, N_CHUNKS // NBUF, outer, 0)
        for b in range(NBUF):
            scat_wait(b)
        plsc.subcore_barrier()

        ocol = (h + 1) * D_FEAT_C + fcol

        @pl.when(s < 15)
        def _():
            pltpu.sync_copy(acc.at[pl.ds(s * W_SMALL, W_SMALL)],
                            cur.at[pl.ds(row_off + s * W_SMALL, W_SMALL)])
            pltpu.sync_copy(acc.at[pl.ds(s * W_SMALL, W_SMALL)],
                            out2d.at[pl.ds(s * W_SMALL, W_SMALL),
                                     pl.ds(ocol, HALF)])

        @pl.when(s == 15)
        def _():
            pltpu.sync_copy(acc.at[pl.ds(15 * W_SMALL, W_LAST)],
                            cur.at[pl.ds(row_off + 15 * W_SMALL, W_LAST)])
            pltpu.sync_copy(acc.at[pl.ds(15 * W_SMALL, W_LAST)],
                            out2d.at[pl.ds(15 * W_SMALL, W_LAST),
                                     pl.ds(ocol, HALF)])

        plsc.subcore_barrier()
        return 0

    lax.fori_loop(0, N_HOPS_C, hop_body, 0)


@jax.jit
def _sc_call(embed, packed, trend, zeros):
    out_t = [
        jax.ShapeDtypeStruct((N_NODES_C, (N_HOPS_C + 1) * D_FEAT_C),
                             jnp.float32),
        jax.ShapeDtypeStruct((2 * N_NODES_C, HALF), jnp.float32),
    ]
    mesh = plsc.VectorSubcoreMesh(core_axis_name="c", subcore_axis_name="s")
    f = pl.kernel(
        _sc_body,
        out_type=out_t,
        mesh=mesh,
        compiler_params=pltpu.CompilerParams(use_tc_tiling_on_sc=False),
        scratch_types=[
            pltpu.VMEM_SHARED((N_ACC, HALF), jnp.float32),
            pltpu.VMEM((NBUF, CHUNK, HALF), jnp.float32),
            pltpu.VMEM((NBUF, 2 * CHUNK), jnp.int32),
            pltpu.VMEM((NBUF, CHUNK), jnp.float32),
            pltpu.SemaphoreType.DMA((NBUF,)),
            pltpu.SemaphoreType.DMA((NBUF,)),
            pltpu.SemaphoreType.DMA((NBUF,)),
        ],
    )
    return f(embed, packed, trend, zeros)


def kernel(embed, adj_sp_norm, edge_index, edge_weight, trend):
    del adj_sp_norm, edge_weight
    row = edge_index[0].astype(jnp.int32)
    col = edge_index[1].astype(jnp.int32)
    per_tile = N_EDGES_C // N_SUBCORES
    pad = EDGES_PER_TILE - per_tile
    row_t = jnp.pad(row.reshape(N_SUBCORES, per_tile), ((0, 0), (0, pad)))
    col_t = jnp.pad(col.reshape(N_SUBCORES, per_tile), ((0, 0), (0, pad)),
                    constant_values=PAD_COL)
    trend_t = jnp.pad(trend.reshape(N_SUBCORES, per_tile),
                      ((0, 0), (0, pad)))
    packed = jnp.concatenate(
        [row_t.reshape(-1, CHUNK), col_t.reshape(-1, CHUNK)], axis=1)
    zeros = jnp.zeros((W_LAST, HALF), jnp.float32)
    out2d, _ = _sc_call(embed, packed, trend_t.reshape(-1), zeros)
    return out2d.reshape(N_NODES_C, N_HOPS_C + 1, D_FEAT_C)


# speedup vs baseline: 2.3080x; 2.3080x over previous
import jax
import jax.numpy as jnp
from jax import lax
from jax.experimental import pallas as pl
from jax.experimental.pallas import tpu as pltpu
from jax.experimental.pallas import tpu_sc as plsc

N_NODES_C = 10000
N_EDGES_C = 320000
D_FEAT_C = 128
N_HOPS_C = 3

HALF = D_FEAT_C // 2
N_SUBCORES = 16
EDGES_PER_TILE = N_EDGES_C // N_SUBCORES
CHUNK = 80
N_CHUNKS = EDGES_PER_TILE // CHUNK
NBUF = 5
G_AHEAD = 2
SCAT_LAG = 3
W_SMALL = 624
W_LAST = N_NODES_C - 15 * W_SMALL


def _sc_body(embed_cat, row2d, col2d, trend2d, zeros_hbm,
             out1, out2, out3, cur,
             acc, gbufs, rowall, colall, tall,
             sem_pre, sem_g, sem_sc):
    c = lax.axis_index("c")
    s = lax.axis_index("s")
    row_off = c * N_NODES_C

    outs = [out1, out2, out3]

    r0 = s * N_CHUNKS
    pltpu.make_async_copy(row2d.at[pl.ds(r0, N_CHUNKS)], rowall,
                          sem_pre).start()
    pltpu.make_async_copy(col2d.at[pl.ds(r0, N_CHUNKS)], colall,
                          sem_pre).start()
    pltpu.make_async_copy(trend2d.at[pl.ds(r0, N_CHUNKS)], tall,
                          sem_pre).start()

    @pl.when(s < 15)
    def _():
        pltpu.sync_copy(embed_cat.at[pl.ds(row_off + s * W_SMALL, W_SMALL)],
                        cur.at[pl.ds(row_off + s * W_SMALL, W_SMALL)])

    @pl.when(s == 15)
    def _():
        pltpu.sync_copy(embed_cat.at[pl.ds(row_off + 15 * W_SMALL, W_LAST)],
                        cur.at[pl.ds(row_off + 15 * W_SMALL, W_LAST)])

    for _ in range(3):
        pltpu.make_async_copy(row2d.at[pl.ds(0, N_CHUNKS)], rowall,
                              sem_pre).wait()

    def rowfix_row(r, _):
        for v in range(CHUNK // 16):
            rowall[r, pl.ds(v * 16, 16)] = (
                rowall[r, pl.ds(v * 16, 16)] + row_off)
        return 0

    lax.fori_loop(0, N_CHUNKS, rowfix_row, 0)

    def gather_start(q, b):
        pltpu.make_async_copy(cur.at[rowall.at[q]],
                              gbufs.at[b], sem_g.at[b]).start()

    def gather_wait(b):
        pltpu.make_async_copy(cur.at[rowall.at[0]],
                              gbufs.at[b], sem_g.at[b]).wait()

    def scat_start(q, b):
        pltpu.make_async_copy(gbufs.at[b], acc.at[colall.at[q]],
                              sem_sc.at[b]).start(add=True)

    def scat_wait(b):
        pltpu.make_async_copy(gbufs.at[b], acc.at[colall.at[0]],
                              sem_sc.at[b]).wait()

    def make_scale(b, q):
        def scale_group(g, _):
            tv16 = tall[q, pl.ds(g * 16, 16)]
            for sub in range(2):
                e0 = g * 16 + sub * 8
                tvs = [jnp.full((16,), tv16[sub * 8 + l], jnp.float32)
                       for l in range(8)]
                vs = [[gbufs[b, e0 + l, pl.ds(jj * 16, 16)]
                       for jj in range(HALF // 16)] for l in range(8)]
                for l in range(8):
                    for jj in range(HALF // 16):
                        gbufs[b, e0 + l, pl.ds(jj * 16, 16)] = (
                            vs[l][jj] * tvs[l])
            return 0
        return scale_group

    def hop_body(h, _):
        @pl.when(s < 15)
        def _():
            pltpu.sync_copy(zeros_hbm.at[pl.ds(0, W_SMALL)],
                            acc.at[pl.ds(s * W_SMALL, W_SMALL)])

        @pl.when(s == 15)
        def _():
            pltpu.sync_copy(zeros_hbm, acc.at[pl.ds(15 * W_SMALL, W_LAST)])

        plsc.subcore_barrier()

        for p in range(G_AHEAD):
            gather_start(p, p)

        def outer(jo, _):
            for b in range(NBUF):
                q = jo * NBUF + b
                bg = (b + G_AHEAD) % NBUF

                @pl.when(q >= SCAT_LAG)
                def _():
                    scat_wait(bg)

                @pl.when(q + G_AHEAD < N_CHUNKS)
                def _():
                    gather_start(q + G_AHEAD, bg)

                gather_wait(b)
                lax.fori_loop(0, CHUNK // 16, make_scale(b, q), 0)
                scat_start(q, b)
            return 0

        lax.fori_loop(0, N_CHUNKS // NBUF, outer, 0)
        for b in range(NBUF - SCAT_LAG, NBUF):
            scat_wait(b)
        plsc.subcore_barrier()

        def write_out(dst):
            @pl.when(s < 15)
            def _():
                pltpu.sync_copy(
                    acc.at[pl.ds(s * W_SMALL, W_SMALL)],
                    dst.at[pl.ds(row_off + s * W_SMALL, W_SMALL)])

            @pl.when(s == 15)
            def _():
                pltpu.sync_copy(
                    acc.at[pl.ds(15 * W_SMALL, W_LAST)],
                    dst.at[pl.ds(row_off + 15 * W_SMALL, W_LAST)])

        write_out(cur)
        for hh, out in enumerate(outs):
            @pl.when(h == hh)
            def _():
                write_out(out)
        plsc.subcore_barrier()
        return 0

    lax.fori_loop(0, N_HOPS_C, hop_body, 0)


@jax.jit
def _sc_call(embed_cat, row2d, col2d, trend2d, zeros):
    out_t = [jax.ShapeDtypeStruct((2 * N_NODES_C, HALF), jnp.float32)] * (
        N_HOPS_C + 1)
    mesh = plsc.VectorSubcoreMesh(core_axis_name="c", subcore_axis_name="s")
    f = pl.kernel(
        _sc_body,
        out_type=out_t,
        mesh=mesh,
        compiler_params=pltpu.CompilerParams(use_tc_tiling_on_sc=False),
        scratch_types=[
            pltpu.VMEM_SHARED((N_NODES_C, HALF), jnp.float32),
            pltpu.VMEM((NBUF, CHUNK, HALF), jnp.float32),
            pltpu.VMEM((N_CHUNKS, CHUNK), jnp.int32),
            pltpu.VMEM((N_CHUNKS, CHUNK), jnp.int32),
            pltpu.VMEM((N_CHUNKS, CHUNK), jnp.float32),
            pltpu.SemaphoreType.DMA,
            pltpu.SemaphoreType.DMA((NBUF,)),
            pltpu.SemaphoreType.DMA((NBUF,)),
        ],
    )
    return f(embed_cat, row2d, col2d, trend2d, zeros)


def kernel(embed, adj_sp_norm, edge_index, edge_weight, trend):
    del adj_sp_norm, edge_weight
    row = edge_index[0].astype(jnp.int32)
    col = edge_index[1].astype(jnp.int32)
    embed_cat = jnp.concatenate([embed[:, :HALF], embed[:, HALF:]], axis=0)
    zeros = jnp.zeros((W_LAST, HALF), jnp.float32)
    out1, out2, out3, _ = _sc_call(
        embed_cat, row.reshape(-1, CHUNK), col.reshape(-1, CHUNK),
        trend.reshape(-1, CHUNK), zeros)

    def unsplit(o):
        return jnp.concatenate([o[:N_NODES_C], o[N_NODES_C:]], axis=1)

    return jnp.stack(
        [embed, unsplit(out1), unsplit(out2), unsplit(out3)], axis=1)


# speedup vs baseline: 2.3381x; 1.0130x over previous
import jax
import jax.numpy as jnp
from jax import lax
from jax.experimental import pallas as pl
from jax.experimental.pallas import tpu as pltpu
from jax.experimental.pallas import tpu_sc as plsc

N_NODES_C = 10000
N_EDGES_C = 320000
D_FEAT_C = 128
N_HOPS_C = 3

HALF = D_FEAT_C // 2
N_SUBCORES = 16
EDGES_PER_TILE = N_EDGES_C // N_SUBCORES
CHUNK = 80
N_CHUNKS = EDGES_PER_TILE // CHUNK
NBUF = 5
G_AHEAD = 3
SCAT_LAG = 2
W_SMALL = 624
W_LAST = N_NODES_C - 15 * W_SMALL


def _sc_body(embed_cat, row2d, col2d, trend2d, zeros_hbm,
             out1, out2, out3, cur,
             acc, gbufs, rowall, colall, tall,
             sem_pre, sem_g, sem_sc):
    c = lax.axis_index("c")
    s = lax.axis_index("s")
    row_off = c * N_NODES_C

    outs = [out1, out2, out3]

    r0 = s * N_CHUNKS
    pltpu.make_async_copy(row2d.at[pl.ds(r0, N_CHUNKS)], rowall,
                          sem_pre).start()
    pltpu.make_async_copy(col2d.at[pl.ds(r0, N_CHUNKS)], colall,
                          sem_pre).start()
    pltpu.make_async_copy(trend2d.at[pl.ds(r0, N_CHUNKS)], tall,
                          sem_pre).start()

    @pl.when(s < 15)
    def _():
        pltpu.sync_copy(embed_cat.at[pl.ds(row_off + s * W_SMALL, W_SMALL)],
                        cur.at[pl.ds(row_off + s * W_SMALL, W_SMALL)])

    @pl.when(s == 15)
    def _():
        pltpu.sync_copy(embed_cat.at[pl.ds(row_off + 15 * W_SMALL, W_LAST)],
                        cur.at[pl.ds(row_off + 15 * W_SMALL, W_LAST)])

    for _ in range(3):
        pltpu.make_async_copy(row2d.at[pl.ds(0, N_CHUNKS)], rowall,
                              sem_pre).wait()

    def rowfix_row(r, _):
        for v in range(CHUNK // 16):
            rowall[r, pl.ds(v * 16, 16)] = (
                rowall[r, pl.ds(v * 16, 16)] + row_off)
        return 0

    lax.fori_loop(0, N_CHUNKS, rowfix_row, 0)

    def gather_start(q, b):
        pltpu.make_async_copy(cur.at[rowall.at[q]],
                              gbufs.at[b], sem_g.at[b]).start()

    def gather_wait(b):
        pltpu.make_async_copy(cur.at[rowall.at[0]],
                              gbufs.at[b], sem_g.at[b]).wait()

    def scat_start(q, b):
        pltpu.make_async_copy(gbufs.at[b], acc.at[colall.at[q]],
                              sem_sc.at[b]).start(add=True)

    def scat_wait(b):
        pltpu.make_async_copy(gbufs.at[b], acc.at[colall.at[0]],
                              sem_sc.at[b]).wait()

    def make_scale(b, q):
        def scale_group(g, _):
            tv16 = tall[q, pl.ds(g * 16, 16)]
            for sub in range(2):
                e0 = g * 16 + sub * 8
                tvs = [jnp.full((16,), tv16[sub * 8 + l], jnp.float32)
                       for l in range(8)]
                vs = [[gbufs[b, e0 + l, pl.ds(jj * 16, 16)]
                       for jj in range(HALF // 16)] for l in range(8)]
                for l in range(8):
                    for jj in range(HALF // 16):
                        gbufs[b, e0 + l, pl.ds(jj * 16, 16)] = (
                            vs[l][jj] * tvs[l])
            return 0
        return scale_group

    def hop_body(h, _):
        @pl.when(s < 15)
        def _():
            pltpu.sync_copy(zeros_hbm.at[pl.ds(0, W_SMALL)],
                            acc.at[pl.ds(s * W_SMALL, W_SMALL)])

        @pl.when(s == 15)
        def _():
            pltpu.sync_copy(zeros_hbm, acc.at[pl.ds(15 * W_SMALL, W_LAST)])

        plsc.subcore_barrier()

        for p in range(G_AHEAD):
            gather_start(p, p)

        def outer(jo, _):
            for b in range(NBUF):
                q = jo * NBUF + b
                bg = (b + G_AHEAD) % NBUF

                @pl.when(q >= SCAT_LAG)
                def _():
                    scat_wait(bg)

                @pl.when(q + G_AHEAD < N_CHUNKS)
                def _():
                    gather_start(q + G_AHEAD, bg)

                gather_wait(b)
                lax.fori_loop(0, CHUNK // 16, make_scale(b, q), 0)
                scat_start(q, b)
            return 0

        lax.fori_loop(0, N_CHUNKS // NBUF, outer, 0)
        for b in range(NBUF - SCAT_LAG, NBUF):
            scat_wait(b)
        plsc.subcore_barrier()

        def write_out(dst):
            @pl.when(s < 15)
            def _():
                pltpu.sync_copy(
                    acc.at[pl.ds(s * W_SMALL, W_SMALL)],
                    dst.at[pl.ds(row_off + s * W_SMALL, W_SMALL)])

            @pl.when(s == 15)
            def _():
                pltpu.sync_copy(
                    acc.at[pl.ds(15 * W_SMALL, W_LAST)],
                    dst.at[pl.ds(row_off + 15 * W_SMALL, W_LAST)])

        write_out(cur)
        for hh, out in enumerate(outs):
            @pl.when(h == hh)
            def _():
                write_out(out)
        plsc.subcore_barrier()
        return 0

    lax.fori_loop(0, N_HOPS_C, hop_body, 0)


@jax.jit
def _sc_call(embed_cat, row2d, col2d, trend2d, zeros):
    out_t = [jax.ShapeDtypeStruct((2 * N_NODES_C, HALF), jnp.float32)] * (
        N_HOPS_C + 1)
    mesh = plsc.VectorSubcoreMesh(core_axis_name="c", subcore_axis_name="s")
    f = pl.kernel(
        _sc_body,
        out_type=out_t,
        mesh=mesh,
        compiler_params=pltpu.CompilerParams(use_tc_tiling_on_sc=False),
        scratch_types=[
            pltpu.VMEM_SHARED((N_NODES_C, HALF), jnp.float32),
            pltpu.VMEM((NBUF, CHUNK, HALF), jnp.float32),
            pltpu.VMEM((N_CHUNKS, CHUNK), jnp.int32),
            pltpu.VMEM((N_CHUNKS, CHUNK), jnp.int32),
            pltpu.VMEM((N_CHUNKS, CHUNK), jnp.float32),
            pltpu.SemaphoreType.DMA,
            pltpu.SemaphoreType.DMA((NBUF,)),
            pltpu.SemaphoreType.DMA((NBUF,)),
        ],
    )
    return f(embed_cat, row2d, col2d, trend2d, zeros)


def kernel(embed, adj_sp_norm, edge_index, edge_weight, trend):
    del adj_sp_norm, edge_weight
    row = edge_index[0].astype(jnp.int32)
    col = edge_index[1].astype(jnp.int32)
    embed_cat = jnp.concatenate([embed[:, :HALF], embed[:, HALF:]], axis=0)
    zeros = jnp.zeros((W_LAST, HALF), jnp.float32)
    out1, out2, out3, _ = _sc_call(
        embed_cat, row.reshape(-1, CHUNK), col.reshape(-1, CHUNK),
        trend.reshape(-1, CHUNK), zeros)

    def unsplit(o):
        return jnp.concatenate([o[:N_NODES_C], o[N_NODES_C:]], axis=1)

    return jnp.stack(
        [embed, unsplit(out1), unsplit(out2), unsplit(out3)], axis=1)


# speedup vs baseline: 2.3467x; 1.0037x over previous
import jax
import jax.numpy as jnp
from jax import lax
from jax.experimental import pallas as pl
from jax.experimental.pallas import tpu as pltpu
from jax.experimental.pallas import tpu_sc as plsc

N_NODES_C = 10000
N_EDGES_C = 320000
D_FEAT_C = 128
N_HOPS_C = 3

HALF = D_FEAT_C // 2
N_SUBCORES = 16
EDGES_PER_TILE = N_EDGES_C // N_SUBCORES
CHUNK = 80
N_CHUNKS = EDGES_PER_TILE // CHUNK
NBUF = 5
G_AHEAD = 3
SCAT_LAG = 2
W_SMALL = 624
W_LAST = N_NODES_C - 15 * W_SMALL


def _sc_body(embed_cat, row2d, col2d, trend2d, zeros_hbm,
             out1, out2, out3, cur,
             acc, gbufs, rowall, colall, tall,
             sem_pre, sem_g, sem_sc):
    c = lax.axis_index("c")
    s = lax.axis_index("s")
    row_off = c * N_NODES_C

    outs = [out1, out2, out3]

    r0 = s * N_CHUNKS
    pltpu.make_async_copy(row2d.at[pl.ds(r0, N_CHUNKS)], rowall,
                          sem_pre).start()
    pltpu.make_async_copy(col2d.at[pl.ds(r0, N_CHUNKS)], colall,
                          sem_pre).start()
    pltpu.make_async_copy(trend2d.at[pl.ds(r0, N_CHUNKS)], tall,
                          sem_pre).start()

    @pl.when(s < 15)
    def _():
        pltpu.sync_copy(embed_cat.at[pl.ds(row_off + s * W_SMALL, W_SMALL)],
                        cur.at[pl.ds(row_off + s * W_SMALL, W_SMALL)])

    @pl.when(s == 15)
    def _():
        pltpu.sync_copy(embed_cat.at[pl.ds(row_off + 15 * W_SMALL, W_LAST)],
                        cur.at[pl.ds(row_off + 15 * W_SMALL, W_LAST)])

    for _ in range(3):
        pltpu.make_async_copy(row2d.at[pl.ds(0, N_CHUNKS)], rowall,
                              sem_pre).wait()

    def rowfix_row(r, _):
        for v in range(CHUNK // 16):
            rowall[r, pl.ds(v * 16, 16)] = (
                rowall[r, pl.ds(v * 16, 16)] + row_off)
        return 0

    lax.fori_loop(0, N_CHUNKS, rowfix_row, 0)

    @pl.when(s < 15)
    def _():
        pltpu.sync_copy(zeros_hbm.at[pl.ds(0, W_SMALL)],
                        acc.at[pl.ds(s * W_SMALL, W_SMALL)])

    @pl.when(s == 15)
    def _():
        pltpu.sync_copy(zeros_hbm, acc.at[pl.ds(15 * W_SMALL, W_LAST)])

    plsc.subcore_barrier()

    def gather_start(q, b):
        pltpu.make_async_copy(cur.at[rowall.at[q]],
                              gbufs.at[b], sem_g.at[b]).start()

    def gather_wait(b):
        pltpu.make_async_copy(cur.at[rowall.at[0]],
                              gbufs.at[b], sem_g.at[b]).wait()

    def scat_start(q, b):
        pltpu.make_async_copy(gbufs.at[b], acc.at[colall.at[q]],
                              sem_sc.at[b]).start(add=True)

    def scat_wait(b):
        pltpu.make_async_copy(gbufs.at[b], acc.at[colall.at[0]],
                              sem_sc.at[b]).wait()

    def make_scale(b, q):
        def scale_group(g, _):
            tv16 = tall[q, pl.ds(g * 16, 16)]
            for sub in range(2):
                e0 = g * 16 + sub * 8
                tvs = [jnp.full((16,), tv16[sub * 8 + l], jnp.float32)
                       for l in range(8)]
                vs = [[gbufs[b, e0 + l, pl.ds(jj * 16, 16)]
                       for jj in range(HALF // 16)] for l in range(8)]
                for l in range(8):
                    for jj in range(HALF // 16):
                        gbufs[b, e0 + l, pl.ds(jj * 16, 16)] = (
                            vs[l][jj] * tvs[l])
            return 0
        return scale_group

    def zero_my_slice():
        @pl.when(s < 15)
        def _():
            pltpu.sync_copy(zeros_hbm.at[pl.ds(0, W_SMALL)],
                            acc.at[pl.ds(s * W_SMALL, W_SMALL)])

        @pl.when(s == 15)
        def _():
            pltpu.sync_copy(zeros_hbm, acc.at[pl.ds(15 * W_SMALL, W_LAST)])

    def hop_body(h, _):
        for p in range(G_AHEAD):
            gather_start(p, p)

        def outer(jo, _):
            for b in range(NBUF):
                q = jo * NBUF + b
                bg = (b + G_AHEAD) % NBUF

                @pl.when(q >= SCAT_LAG)
                def _():
                    scat_wait(bg)

                @pl.when(q + G_AHEAD < N_CHUNKS)
                def _():
                    gather_start(q + G_AHEAD, bg)

                gather_wait(b)
                lax.fori_loop(0, CHUNK // 16, make_scale(b, q), 0)
                scat_start(q, b)
            return 0

        lax.fori_loop(0, N_CHUNKS // NBUF, outer, 0)
        for b in range(NBUF - SCAT_LAG, NBUF):
            scat_wait(b)
        plsc.subcore_barrier()

        def wout_start(dst):
            @pl.when(s < 15)
            def _():
                pltpu.make_async_copy(
                    acc.at[pl.ds(s * W_SMALL, W_SMALL)],
                    dst.at[pl.ds(row_off + s * W_SMALL, W_SMALL)],
                    sem_pre).start()

            @pl.when(s == 15)
            def _():
                pltpu.make_async_copy(
                    acc.at[pl.ds(15 * W_SMALL, W_LAST)],
                    dst.at[pl.ds(row_off + 15 * W_SMALL, W_LAST)],
                    sem_pre).start()

        def wout_wait(dst):
            @pl.when(s < 15)
            def _():
                pltpu.make_async_copy(
                    acc.at[pl.ds(s * W_SMALL, W_SMALL)],
                    dst.at[pl.ds(row_off + s * W_SMALL, W_SMALL)],
                    sem_pre).wait()

            @pl.when(s == 15)
            def _():
                pltpu.make_async_copy(
                    acc.at[pl.ds(15 * W_SMALL, W_LAST)],
                    dst.at[pl.ds(row_off + 15 * W_SMALL, W_LAST)],
                    sem_pre).wait()

        wout_start(cur)
        for hh, out in enumerate(outs):
            @pl.when(h == hh)
            def _():
                wout_start(out)
        wout_wait(cur)
        wout_wait(cur)
        zero_my_slice()
        plsc.subcore_barrier()
        return 0

    lax.fori_loop(0, N_HOPS_C, hop_body, 0)


@jax.jit
def _sc_call(embed_cat, row2d, col2d, trend2d, zeros):
    out_t = [jax.ShapeDtypeStruct((2 * N_NODES_C, HALF), jnp.float32)] * (
        N_HOPS_C + 1)
    mesh = plsc.VectorSubcoreMesh(core_axis_name="c", subcore_axis_name="s")
    f = pl.kernel(
        _sc_body,
        out_type=out_t,
        mesh=mesh,
        compiler_params=pltpu.CompilerParams(use_tc_tiling_on_sc=False),
        scratch_types=[
            pltpu.VMEM_SHARED((N_NODES_C, HALF), jnp.float32),
            pltpu.VMEM((NBUF, CHUNK, HALF), jnp.float32),
            pltpu.VMEM((N_CHUNKS, CHUNK), jnp.int32),
            pltpu.VMEM((N_CHUNKS, CHUNK), jnp.int32),
            pltpu.VMEM((N_CHUNKS, CHUNK), jnp.float32),
            pltpu.SemaphoreType.DMA,
            pltpu.SemaphoreType.DMA((NBUF,)),
            pltpu.SemaphoreType.DMA((NBUF,)),
        ],
    )
    return f(embed_cat, row2d, col2d, trend2d, zeros)


def kernel(embed, adj_sp_norm, edge_index, edge_weight, trend):
    del adj_sp_norm, edge_weight
    row = edge_index[0].astype(jnp.int32)
    col = edge_index[1].astype(jnp.int32)
    embed_cat = jnp.concatenate([embed[:, :HALF], embed[:, HALF:]], axis=0)
    zeros = jnp.zeros((W_LAST, HALF), jnp.float32)
    out1, out2, out3, _ = _sc_call(
        embed_cat, row.reshape(-1, CHUNK), col.reshape(-1, CHUNK),
        trend.reshape(-1, CHUNK), zeros)

    def unsplit(o):
        return jnp.concatenate([o[:N_NODES_C], o[N_NODES_C:]], axis=1)

    return jnp.stack(
        [embed, unsplit(out1), unsplit(out2), unsplit(out3)], axis=1)
